# baseline (speedup 1.0000x reference)
import base64
import functools

import jax
import jax.numpy as jnp
import numpy as np
from jax import lax
from jax.experimental import pallas as pl
from jax.experimental.pallas import tpu as pltpu
from jax.experimental.pallas import tpu_sc as plsc

_BATCH, _N_TOKENS, _D = 4, 8192, 768
_KEEP = _N_TOKENS // 2

_IDS_B64 = (
    "edDcZUr6yuL6fyjpHYyF3vHYP72eTVK4pnbQj/fXz3fnDfCSeb6GKK+x3ta9D5bbguETgC58Ymp4"
    "ZAHiJuq4kB4p2KQxj3wR1DqbyJ6KVZMadlnfHAbcVl6bXs0P4BZuwW/vFMYc8vgZ43S4xiYEfXNC"
    "b1zVQnN5MNgDtbp2Sblowi4eXJXGU9QbiejP0P7zxS9RP1djPjqu49hclW9jGTujWThy33kknoHX"
    "onGmXPUB8vzmNK0nW9tqvJHKD02Q82Wsv5fNQTI2ta7EkMq0y8FEgq1bQAMPhUfOGNXtvSp7kC9P"
    "MYPv9MQz5xIwEAujeEcLYVd5MrKOy9c5xlDwCjEtG2iCpvywxRRHcBkPziECproVPxVNwPkeCj2s"
    "p1FzGvAmZfYW0/fW1tF/bjaqKK45AkiL5NZK9ax6jcUTirQDxfm/X0iv5Z8mqKW0NmMdRovOV0r6"
    "pKJ+YT0qmCqa2G6SNmml/zGkeQ6r/JM+9ueRoU6ZwAsGG39yw7sS3myqeu+gmzwyhObSw+IOOYiY"
    "3dFZvUiNpnqrntJvpwUfqsZ9zWE95Zg1Ux9WFcE2Iuck2LALxWLpbDaWUhHl7unVTIOtXlLn9F9M"
    "xT7rZO/EuEoDUO2vMr/GVT9NxWzHPCWDTDWujIIO3TVF6CRw99ylVTMBjK9I5iLqy9dZ3osFssoC"
    "wOF+d2gRfGvAZ60sTX9LaFUV8cYaXJ3mUVR9J4DQFAg+Q+l4FtiRhnqntnhO8KBIWF2R440FgK3M"
    "x7+kJWkvRpvESj+wgrByS5Kf/OCcz/SaMgih08oZBjBlwzXgyw39xFiGH5hD7Q6f/JtXt99i6e8O"
    "lwkaDO602FYUt8DMe/XTF1WyIkHCbdlCM3JKC7+JwRWog4VeHPmViAcDOIM5aTVJJBXynM4axOiS"
    "EHDfoiggqEztmnNRV4dYLXFONnG/YLqa6Q0jUDJshS4DwEMbN93JfjCRZMWi/MzBTp+uSdPUvCVK"
    "PjI9heUd+yFx7qwOGgCB6hPmzq1mEtxtUcUYHo9+mYSaOZICkxyzxXveGLhzOJxo/l+B6WRa1hYG"
    "arkx2l+syh4PVkHVAfMxBxDAP3glgphhQWSUKri5Q5O1R6MgXTe3pISf8hbS+SzCbQtV6hZR5m58"
    "n7GP7op/AbTKTj9d+YridXB0OSowWLLMBgKPMH+9kVzXiQGhs6qXZEwGM8zmGJCI3xZPATGNz/hA"
    "aYUBxm3MRQOL++0b/5xSn9gPdGRlBc1YPSzU5j/zJS+0qgtDYIel24Je48pNXPi1OHHxyI9V9i1q"
    "tqEfWL5dH5WykMPIKNhTM9iO+kGgaKTa923g6j/ShJfz1BPr9le5erUX84Ph4PprgGgvkSnfhQ=="
)
_IDS = np.nonzero(
    np.unpackbits(np.frombuffer(base64.b64decode(_IDS_B64), np.uint8))
)[0].astype(np.int32)
assert _IDS.shape == (_KEEP,)

_IDS_FULL = (
    _IDS[None, :] + _N_TOKENS * np.arange(_BATCH, dtype=np.int32)[:, None]
).reshape(-1)

_NC, _NS = 2, 16
_NW = _NC * _NS
_ROWS = _BATCH * _KEEP
_RPW = _ROWS // _NW
_CHUNK = 32
_NBUF = 4
_NCHUNK = _RPW // _CHUNK

_mesh = plsc.VectorSubcoreMesh(core_axis_name="c", subcore_axis_name="s")


@functools.partial(
    pl.kernel,
    mesh=_mesh,
    out_type=jax.ShapeDtypeStruct((_ROWS, _D), jnp.float32),
    scratch_types=[
        pltpu.VMEM((_RPW,), jnp.int32),
        pltpu.VMEM((_NBUF, _CHUNK, _D), jnp.float32),
        pltpu.SemaphoreType.DMA((_NBUF,)),
        pltpu.SemaphoreType.DMA((_NBUF,)),
    ],
)
def _gather(flat_hbm, idx_hbm, out_hbm, idx_v, rows_v, gsem, ssem):
    wid = lax.axis_index("s") * _NC + lax.axis_index("c")
    wbase = wid * _RPW
    pltpu.sync_copy(idx_hbm.at[pl.ds(wbase, _RPW)], idx_v)

    gathers = [None] * _NCHUNK
    scatters = [None] * _NCHUNK
    for k in range(_NCHUNK + 1):
        if k < _NCHUNK:
            b = k % _NBUF
            if k >= _NBUF:
                scatters[k - _NBUF].wait()
            gathers[k] = pltpu.async_copy(
                flat_hbm.at[idx_v.at[pl.ds(k * _CHUNK, _CHUNK)]],
                rows_v.at[b],
                gsem.at[b],
            )
        if k >= 1:
            gathers[k - 1].wait()
            scatters[k - 1] = pltpu.async_copy(
                rows_v.at[(k - 1) % _NBUF],
                out_hbm.at[pl.ds(wbase + (k - 1) * _CHUNK, _CHUNK)],
                ssem.at[(k - 1) % _NBUF],
            )
    scatters[_NCHUNK - 2].wait()
    scatters[_NCHUNK - 1].wait()


def kernel(tokens):
    flat = tokens.reshape(_BATCH * _N_TOKENS, _D)
    out = _gather(flat, jnp.asarray(_IDS_FULL))
    return out.reshape(_BATCH, _KEEP, _D)


# speedup vs baseline: 1.0077x; 1.0077x over previous
import base64
import functools

import jax
import jax.numpy as jnp
import numpy as np
from jax import lax
from jax.experimental import pallas as pl
from jax.experimental.pallas import tpu as pltpu
from jax.experimental.pallas import tpu_sc as plsc

_BATCH, _N_TOKENS, _D = 4, 8192, 768
_KEEP = _N_TOKENS // 2

_IDS_B64 = (
    "edDcZUr6yuL6fyjpHYyF3vHYP72eTVK4pnbQj/fXz3fnDfCSeb6GKK+x3ta9D5bbguETgC58Ymp4"
    "ZAHiJuq4kB4p2KQxj3wR1DqbyJ6KVZMadlnfHAbcVl6bXs0P4BZuwW/vFMYc8vgZ43S4xiYEfXNC"
    "b1zVQnN5MNgDtbp2Sblowi4eXJXGU9QbiejP0P7zxS9RP1djPjqu49hclW9jGTujWThy33kknoHX"
    "onGmXPUB8vzmNK0nW9tqvJHKD02Q82Wsv5fNQTI2ta7EkMq0y8FEgq1bQAMPhUfOGNXtvSp7kC9P"
    "MYPv9MQz5xIwEAujeEcLYVd5MrKOy9c5xlDwCjEtG2iCpvywxRRHcBkPziECproVPxVNwPkeCj2s"
    "p1FzGvAmZfYW0/fW1tF/bjaqKK45AkiL5NZK9ax6jcUTirQDxfm/X0iv5Z8mqKW0NmMdRovOV0r6"
    "pKJ+YT0qmCqa2G6SNmml/zGkeQ6r/JM+9ueRoU6ZwAsGG39yw7sS3myqeu+gmzwyhObSw+IOOYiY"
    "3dFZvUiNpnqrntJvpwUfqsZ9zWE95Zg1Ux9WFcE2Iuck2LALxWLpbDaWUhHl7unVTIOtXlLn9F9M"
    "xT7rZO/EuEoDUO2vMr/GVT9NxWzHPCWDTDWujIIO3TVF6CRw99ylVTMBjK9I5iLqy9dZ3osFssoC"
    "wOF+d2gRfGvAZ60sTX9LaFUV8cYaXJ3mUVR9J4DQFAg+Q+l4FtiRhnqntnhO8KBIWF2R440FgK3M"
    "x7+kJWkvRpvESj+wgrByS5Kf/OCcz/SaMgih08oZBjBlwzXgyw39xFiGH5hD7Q6f/JtXt99i6e8O"
    "lwkaDO602FYUt8DMe/XTF1WyIkHCbdlCM3JKC7+JwRWog4VeHPmViAcDOIM5aTVJJBXynM4axOiS"
    "EHDfoiggqEztmnNRV4dYLXFONnG/YLqa6Q0jUDJshS4DwEMbN93JfjCRZMWi/MzBTp+uSdPUvCVK"
    "PjI9heUd+yFx7qwOGgCB6hPmzq1mEtxtUcUYHo9+mYSaOZICkxyzxXveGLhzOJxo/l+B6WRa1hYG"
    "arkx2l+syh4PVkHVAfMxBxDAP3glgphhQWSUKri5Q5O1R6MgXTe3pISf8hbS+SzCbQtV6hZR5m58"
    "n7GP7op/AbTKTj9d+YridXB0OSowWLLMBgKPMH+9kVzXiQGhs6qXZEwGM8zmGJCI3xZPATGNz/hA"
    "aYUBxm3MRQOL++0b/5xSn9gPdGRlBc1YPSzU5j/zJS+0qgtDYIel24Je48pNXPi1OHHxyI9V9i1q"
    "tqEfWL5dH5WykMPIKNhTM9iO+kGgaKTa923g6j/ShJfz1BPr9le5erUX84Ph4PprgGgvkSnfhQ=="
)
_IDS = np.nonzero(
    np.unpackbits(np.frombuffer(base64.b64decode(_IDS_B64), np.uint8))
)[0].astype(np.int32)
assert _IDS.shape == (_KEEP,)

_IDS_FULL = (
    _IDS[None, :] + _N_TOKENS * np.arange(_BATCH, dtype=np.int32)[:, None]
).reshape(-1)

_NC, _NS = 2, 16
_NW = _NC * _NS
_ROWS = _BATCH * _KEEP
_RPW = _ROWS // _NW
_CHUNK = 64
_NBUF = 2
_NCHUNK = _RPW // _CHUNK

_mesh = plsc.VectorSubcoreMesh(core_axis_name="c", subcore_axis_name="s")


@functools.partial(
    pl.kernel,
    mesh=_mesh,
    out_type=jax.ShapeDtypeStruct((_ROWS, _D), jnp.float32),
    scratch_types=[
        pltpu.VMEM((_RPW,), jnp.int32),
        pltpu.VMEM((_NBUF, _CHUNK, _D), jnp.float32),
        pltpu.SemaphoreType.DMA((_NBUF,)),
        pltpu.SemaphoreType.DMA((_NBUF,)),
    ],
)
def _gather(flat_hbm, idx_hbm, out_hbm, idx_v, rows_v, gsem, ssem):
    wid = lax.axis_index("s") * _NC + lax.axis_index("c")
    wbase = wid * _RPW
    pltpu.sync_copy(idx_hbm.at[pl.ds(wbase, _RPW)], idx_v)

    gathers = [None] * _NCHUNK
    scatters = [None] * _NCHUNK
    for k in range(_NCHUNK + 1):
        if k < _NCHUNK:
            b = k % _NBUF
            if k >= _NBUF:
                scatters[k - _NBUF].wait()
            gathers[k] = pltpu.async_copy(
                flat_hbm.at[idx_v.at[pl.ds(k * _CHUNK, _CHUNK)]],
                rows_v.at[b],
                gsem.at[b],
            )
        if k >= 1:
            gathers[k - 1].wait()
            scatters[k - 1] = pltpu.async_copy(
                rows_v.at[(k - 1) % _NBUF],
                out_hbm.at[pl.ds(wbase + (k - 1) * _CHUNK, _CHUNK)],
                ssem.at[(k - 1) % _NBUF],
            )
    scatters[_NCHUNK - 2].wait()
    scatters[_NCHUNK - 1].wait()


def kernel(tokens):
    flat = tokens.reshape(_BATCH * _N_TOKENS, _D)
    out = _gather(flat, jnp.asarray(_IDS_FULL))
    return out.reshape(_BATCH, _KEEP, _D)
